# baseline (speedup 1.0000x reference)
import functools

import jax
import jax.numpy as jnp
from jax import lax
from jax.experimental import pallas as pl
from jax.experimental.pallas import tpu as pltpu
from jax.experimental.pallas import tpu_sc as plsc

NC = 2
NS = 16
NW = NC * NS
LN_EPS = 1e-5


def _rsqrt16(t):
    i = plsc.bitcast(t, jnp.int32)
    i = jnp.int32(0x5F3759DF) - lax.shift_right_logical(i, 1)
    y = plsc.bitcast(i, jnp.float32)
    ht = t * 0.5
    for _ in range(3):
        y = y * (1.5 - ht * y * y)
    return y


@functools.partial(jax.jit, static_argnums=(5, 6, 7))
def _sc_embed_ln(xf, emb, pos, gamma, beta, B, S, D):
    NV = D // 16
    BW = B // NW
    R = S
    NCH = BW
    NB = 4

    mesh = plsc.VectorSubcoreMesh(core_axis_name="c", subcore_axis_name="s",
                                  num_cores=NC, num_subcores=NS)

    @functools.partial(
        pl.kernel,
        out_type=jax.ShapeDtypeStruct((B * S, D), jnp.float32),
        mesh=mesh,
        compiler_params=pltpu.CompilerParams(needs_layout_passes=False,
                                             use_tc_tiling_on_sc=False),
        scratch_types=[
            pltpu.VMEM((BW * S,), jnp.int32),
            pltpu.VMEM((S, D), jnp.float32),
            pltpu.VMEM((D,), jnp.float32),
            pltpu.VMEM((D,), jnp.float32),
            [pltpu.VMEM((R, D), jnp.float32) for _ in range(NB)],
            [pltpu.SemaphoreType.DMA for _ in range(NB)],
            [pltpu.SemaphoreType.DMA for _ in range(NB)],
        ],
    )
    def k(x_hbm, emb_hbm, pos_hbm, gamma_hbm, beta_hbm, out_hbm,
          idx_v, pos_v, gam_v, bet_v, bufs, gsems, ssems):
        wid = lax.axis_index("s") * NC + lax.axis_index("c")
        row0 = wid * (BW * S)

        pltpu.sync_copy(x_hbm.at[pl.ds(row0, BW * S)], idx_v)
        pltpu.sync_copy(pos_hbm, pos_v)
        pltpu.sync_copy(gamma_hbm, gam_v)
        pltpu.sync_copy(beta_hbm, bet_v)

        def fold(p, _):
            for kk in range(NV):
                sl = pl.ds(16 * kk, 16)
                pos_v[p, sl] = pos_v[p, sl] + bet_v[sl]
            return 0
        lax.fori_loop(0, S, fold, 0)

        def g_src(c):
            return emb_hbm.at[idx_v.at[pl.ds(c * R, R)]]

        def s_dst(c):
            return out_hbm.at[pl.ds(row0 + c * R, R)]

        def start_gather(c, b):
            pltpu.async_copy(g_src(c), bufs[b], gsems[b])

        def wait_gather(c, b):
            pltpu.make_async_copy(g_src(c), bufs[b], gsems[b]).wait()

        def start_store(c, b):
            pltpu.async_copy(bufs[b], s_dst(c), ssems[b])

        def wait_store(c, b):
            pltpu.make_async_copy(bufs[b], s_dst(c), ssems[b]).wait()

        def compute(b):
            buf = bufs[b]

            def p_body(p, _):
                v = [buf[p, pl.ds(16 * kk, 16)] for kk in range(NV)]
                s = (v[0] + v[1]) + (v[2] + v[3])
                q = ((v[0] * v[0] + v[1] * v[1])
                     + (v[2] * v[2] + v[3] * v[3]))
                mu = jnp.full((16,), jnp.sum(s) * (1.0 / D), jnp.float32)
                ex2 = jnp.full((16,), jnp.sum(q) * (1.0 / D), jnp.float32)
                var = ex2 - mu * mu
                rstd = _rsqrt16(var + LN_EPS)
                for kk in range(NV):
                    sl = pl.ds(16 * kk, 16)
                    rg = rstd * gam_v[sl]
                    buf[p, sl] = (v[kk] - mu) * rg + pos_v[p, sl]
                return 0

            lax.fori_loop(0, R, p_body, 0)

        start_gather(0, 0)

        def g_body(g, _):
            for b in range(NB):
                c = g + b

                @pl.when(c >= NB - 1)
                def _():
                    wait_store(c - (NB - 1), (b + 1) % NB)

                @pl.when(c + 1 < NCH)
                def _():
                    start_gather(c + 1, (b + 1) % NB)

                wait_gather(c, b)
                compute(b)
                start_store(c, b)
            return 0

        lax.fori_loop(0, NCH // NB, lambda i, u: g_body(i * NB, u), 0)

        for c in range(NCH - (NB - 1), NCH):
            wait_store(c, c % NB)

    return k(xf, emb, pos, gamma, beta)


def kernel(x, emb_table, pos_table, gamma, beta):
    B, S = x.shape
    D = emb_table.shape[1]
    xf = x.reshape(B * S).astype(jnp.int32)
    pos = lax.slice_in_dim(pos_table, 1, S + 1, axis=0)
    out = _sc_embed_ln(xf, emb_table, pos, gamma, beta, B, S, D)
    return out.reshape(B, S, D)


# speedup vs baseline: 2.0695x; 2.0695x over previous
import functools

import jax
import jax.numpy as jnp
from jax import lax
from jax.experimental import pallas as pl
from jax.experimental.pallas import tpu as pltpu
from jax.experimental.pallas import tpu_sc as plsc

NC = 2
NS = 16
NW = NC * NS
LN_EPS = 1e-5


def _rsqrt16(t):
    i = plsc.bitcast(t, jnp.int32)
    i = jnp.int32(0x5F3759DF) - lax.shift_right_logical(i, 1)
    y = plsc.bitcast(i, jnp.float32)
    ht = t * 0.5
    for _ in range(2):
        y = y * (1.5 - ht * y * y)
    return y


@functools.partial(jax.jit, static_argnums=(5, 6, 7))
def _sc_embed_ln(xf, emb, pos, gamma, beta, B, S, D):
    NV = D // 16
    BW = B // NW
    R = S
    NCH = BW
    NB = 4

    mesh = plsc.VectorSubcoreMesh(core_axis_name="c", subcore_axis_name="s",
                                  num_cores=NC, num_subcores=NS)

    @functools.partial(
        pl.kernel,
        out_type=jax.ShapeDtypeStruct((B * S, D), jnp.float32),
        mesh=mesh,
        compiler_params=pltpu.CompilerParams(needs_layout_passes=False,
                                             use_tc_tiling_on_sc=False),
        scratch_types=[
            pltpu.VMEM((BW * S,), jnp.int32),
            pltpu.VMEM((S, D), jnp.float32),
            pltpu.VMEM((D,), jnp.float32),
            pltpu.VMEM((D,), jnp.float32),
            [pltpu.VMEM((R, D), jnp.float32) for _ in range(NB)],
            [pltpu.SemaphoreType.DMA for _ in range(NB)],
            [pltpu.SemaphoreType.DMA for _ in range(NB)],
        ],
    )
    def k(x_hbm, emb_hbm, pos_hbm, gamma_hbm, beta_hbm, out_hbm,
          idx_v, pos_v, gam_v, bet_v, bufs, gsems, ssems):
        wid = lax.axis_index("s") * NC + lax.axis_index("c")
        row0 = wid * (BW * S)

        pltpu.sync_copy(x_hbm.at[pl.ds(row0, BW * S)], idx_v)
        pltpu.sync_copy(pos_hbm, pos_v)
        pltpu.sync_copy(gamma_hbm, gam_v)
        pltpu.sync_copy(beta_hbm, bet_v)

        def fold(p, _):
            for kk in range(NV):
                sl = pl.ds(16 * kk, 16)
                pos_v[p, sl] = pos_v[p, sl] + bet_v[sl]
            return 0
        lax.fori_loop(0, S, fold, 0)

        def g_src(c):
            return emb_hbm.at[idx_v.at[pl.ds(c * R, R)]]

        def s_dst(c):
            return out_hbm.at[pl.ds(row0 + c * R, R)]

        def start_gather(c, b):
            pltpu.async_copy(g_src(c), bufs[b], gsems[b])

        def wait_gather(c, b):
            pltpu.make_async_copy(g_src(c), bufs[b], gsems[b]).wait()

        def start_store(c, b):
            pltpu.async_copy(bufs[b], s_dst(c), ssems[b])

        def wait_store(c, b):
            pltpu.make_async_copy(bufs[b], s_dst(c), ssems[b]).wait()

        def compute(b):
            buf = bufs[b]

            @plsc.parallel_loop(0, R, unroll=4)
            def _(p):
                v = [buf[p, pl.ds(16 * kk, 16)] for kk in range(NV)]
                s = (v[0] + v[1]) + (v[2] + v[3])
                q = ((v[0] * v[0] + v[1] * v[1])
                     + (v[2] * v[2] + v[3] * v[3]))
                mu = jnp.full((16,), jnp.sum(s) * (1.0 / D), jnp.float32)
                ex2 = jnp.full((16,), jnp.sum(q) * (1.0 / D), jnp.float32)
                var = ex2 - mu * mu
                rstd = _rsqrt16(var + LN_EPS)
                for kk in range(NV):
                    sl = pl.ds(16 * kk, 16)
                    rg = rstd * gam_v[sl]
                    buf[p, sl] = (v[kk] - mu) * rg + pos_v[p, sl]

        start_gather(0, 0)

        def g_body(g, _):
            for b in range(NB):
                c = g + b

                @pl.when(c >= NB - 1)
                def _():
                    wait_store(c - (NB - 1), (b + 1) % NB)

                @pl.when(c + 1 < NCH)
                def _():
                    start_gather(c + 1, (b + 1) % NB)

                wait_gather(c, b)
                compute(b)
                start_store(c, b)
            return 0

        lax.fori_loop(0, NCH // NB, lambda i, u: g_body(i * NB, u), 0)

        for c in range(NCH - (NB - 1), NCH):
            wait_store(c, c % NB)

    return k(xf, emb, pos, gamma, beta)


def kernel(x, emb_table, pos_table, gamma, beta):
    B, S = x.shape
    D = emb_table.shape[1]
    xf = x.reshape(B * S).astype(jnp.int32)
    pos = lax.slice_in_dim(pos_table, 1, S + 1, axis=0)
    out = _sc_embed_ln(xf, emb_table, pos, gamma, beta, B, S, D)
    return out.reshape(B, S, D)
